# baseline (speedup 1.0000x reference)
import jax
import jax.numpy as jnp
from jax import lax
from jax.experimental import pallas as pl
from jax.experimental.pallas import tpu as pltpu
from jax.experimental.pallas import tpu_sc as plsc

NUM_BINS = 256
WORD_L = 20
NUM_CORES = 2
NUM_SUBCORES = 16
NUM_WORKERS = NUM_CORES * NUM_SUBCORES
CHUNK = 128


def _hist_body(ids_hbm, out_hbm, ids_v, hist0_v, hist1_v, ids_sem, out_sems):
    wid = lax.axis_index("s") * NUM_CORES + lax.axis_index("c")
    _, row_words, _ = out_hbm.shape
    num_chunks = row_words // CHUNK
    hists = [hist0_v, hist1_v]

    ones = jnp.ones((16,), jnp.float32)
    zeros_f = jnp.zeros((16,), jnp.float32)
    lanes = lax.iota(jnp.int32, 16)
    word_of_lane = [(lanes + 16 * v) // 20 for v in range(5)]

    ids_dma = pltpu.async_copy(
        ids_hbm.at[pl.ds(wid * row_words * WORD_L, row_words * WORD_L)],
        ids_v,
        ids_sem,
    )

    for h in hists:
        @plsc.parallel_loop(0, CHUNK * (NUM_BINS // 16), unroll=8)
        def _zero(i, h=h):
            h[i >> 4, pl.ds((i & 15) * 16, 16)] = zeros_f

    ids_dma.wait()

    def scatter(c, p):
        hist = hists[p]
        ibase = c * CHUNK * WORD_L

        @plsc.parallel_loop(0, CHUNK // 4, unroll=4)
        def _scatter(j):
            row = ibase + j * (4 * WORD_L)
            r0 = jnp.broadcast_to(j * 4, (16,))
            for v in range(5):
                ids16 = ids_v[pl.ds(row + v * 16, 16)]
                plsc.addupdate_scatter(hist, [r0 + word_of_lane[v], ids16],
                                       ones, mask=ids16 != 0)

    def clear(c, p):
        hist = hists[p]
        ibase = c * CHUNK * WORD_L

        @plsc.parallel_loop(0, CHUNK // 4, unroll=4)
        def _clear(j):
            row = ibase + j * (4 * WORD_L)
            r0 = jnp.broadcast_to(j * 4, (16,))
            for v in range(5):
                ids16 = ids_v[pl.ds(row + v * 16, 16)]
                plsc.store_scatter(hist, [r0 + word_of_lane[v], ids16],
                                   zeros_f, mask=ids16 != 0)

    def start_out(c, p):
        return pltpu.async_copy(
            hists[p],
            out_hbm.at[wid, pl.ds(c * CHUNK, CHUNK), :],
            out_sems.at[p],
        )

    out_dma = [None, None]
    for c in range(num_chunks):
        p = c % 2
        scatter(c, p)
        if out_dma[1 - p] is not None:
            out_dma[1 - p].wait()
            out_dma[p] = start_out(c, p)
            if c + 1 < num_chunks:
                clear(c - 1, 1 - p)
        else:
            out_dma[p] = start_out(c, p)
    out_dma[(num_chunks - 1) % 2].wait()


def kernel(token_ids):
    B, W, L = token_ids.shape
    mesh = plsc.VectorSubcoreMesh(
        core_axis_name="c",
        subcore_axis_name="s",
        num_cores=NUM_CORES,
        num_subcores=NUM_SUBCORES,
    )
    return pl.kernel(
        _hist_body,
        out_type=jax.ShapeDtypeStruct((B, W, NUM_BINS), jnp.float32),
        mesh=mesh,
        scratch_types=[
            pltpu.VMEM((W * L,), jnp.int32),
            pltpu.VMEM((CHUNK, NUM_BINS), jnp.float32),
            pltpu.VMEM((CHUNK, NUM_BINS), jnp.float32),
            pltpu.SemaphoreType.DMA,
            pltpu.SemaphoreType.DMA((2,)),
        ],
        compiler_params=pltpu.CompilerParams(needs_layout_passes=False),
    )(token_ids.reshape(-1))
